# baseline (speedup 1.0000x reference)
import functools

import jax
import jax.numpy as jnp
from jax import lax
from jax.experimental import pallas as pl
from jax.experimental.pallas import tpu as pltpu
from jax.experimental.pallas import tpu_sc as plsc

BATCH = 4096
HIST_LEN = 200
EMBEDDING_DIM = 64
B_TOTAL = BATCH * HIST_LEN

NUM_CORES = 2
NUM_SUBCORES = 16
NUM_WORKERS = NUM_CORES * NUM_SUBCORES
B_PER_W = B_TOTAL // NUM_WORKERS

CHUNK = 512
N_CHUNKS = B_PER_W // CHUNK

_mesh = plsc.VectorSubcoreMesh(core_axis_name="c", subcore_axis_name="s")


@functools.partial(
    pl.kernel,
    out_type=jax.ShapeDtypeStruct((B_TOTAL, EMBEDDING_DIM), jnp.float32),
    mesh=_mesh,
    compiler_params=pltpu.CompilerParams(use_tc_tiling_on_sc=False),
    scratch_types=[
        pltpu.VMEM((B_PER_W,), jnp.int32),
        pltpu.VMEM((CHUNK, EMBEDDING_DIM), jnp.float32),
        pltpu.SemaphoreType.DMA,
    ],
)
def _sc_gather(idx_hbm, table_hbm, out_hbm, idx_v, rows_v, sem):
    wid = lax.axis_index("s") * NUM_CORES + lax.axis_index("c")
    base = wid * B_PER_W
    pltpu.sync_copy(idx_hbm.at[pl.ds(base, B_PER_W)], idx_v)

    def body(i, carry):
        off = i * CHUNK
        pltpu.async_copy(
            table_hbm.at[idx_v.at[pl.ds(off, CHUNK)]], rows_v, sem
        ).wait()
        pltpu.sync_copy(rows_v, out_hbm.at[pl.ds(base + off, CHUNK)])
        return carry

    lax.fori_loop(0, N_CHUNKS, body, 0)


def kernel(token_ids, weights):
    flat_ids = token_ids.reshape(B_TOTAL)
    out = _sc_gather(flat_ids, weights)
    return out.reshape(BATCH, HIST_LEN, EMBEDDING_DIM)


# speedup vs baseline: 1.0235x; 1.0235x over previous
import functools

import jax
import jax.numpy as jnp
from jax import lax
from jax.experimental import pallas as pl
from jax.experimental.pallas import tpu as pltpu
from jax.experimental.pallas import tpu_sc as plsc

BATCH = 4096
HIST_LEN = 200
EMBEDDING_DIM = 64
B_TOTAL = BATCH * HIST_LEN

NUM_CORES = 2
NUM_SUBCORES = 16
NUM_WORKERS = NUM_CORES * NUM_SUBCORES
B_PER_W = B_TOTAL // NUM_WORKERS

CHUNK = 256
N_CHUNKS = B_PER_W // CHUNK
NBUF = 4

_mesh = plsc.VectorSubcoreMesh(core_axis_name="c", subcore_axis_name="s")


@functools.partial(
    pl.kernel,
    out_type=jax.ShapeDtypeStruct((B_TOTAL, EMBEDDING_DIM), jnp.float32),
    mesh=_mesh,
    compiler_params=pltpu.CompilerParams(use_tc_tiling_on_sc=False),
    scratch_types=[
        pltpu.VMEM((B_PER_W,), jnp.int32),
        [pltpu.VMEM((CHUNK, EMBEDDING_DIM), jnp.float32) for _ in range(NBUF)],
        [pltpu.SemaphoreType.DMA for _ in range(NBUF)],
        [pltpu.SemaphoreType.DMA for _ in range(NBUF)],
    ],
)
def _sc_gather(idx_hbm, table_hbm, out_hbm, idx_v, rows, gsem, wsem):
    wid = lax.axis_index("s") * NUM_CORES + lax.axis_index("c")
    base = wid * B_PER_W
    pltpu.sync_copy(idx_hbm.at[pl.ds(base, B_PER_W)], idx_v)

    def gather_copy(i, b):
        return pltpu.make_async_copy(
            table_hbm.at[idx_v.at[pl.ds(i * CHUNK, CHUNK)]], rows[b], gsem[b]
        )

    def write_copy(i, b):
        return pltpu.make_async_copy(
            rows[b], out_hbm.at[pl.ds(base + i * CHUNK, CHUNK)], wsem[b]
        )

    gather_copy(0, 0).start()
    gather_copy(1, 1).start()

    def group(g, carry):
        for b in range(NBUF):
            i = g * NBUF + b
            gather_copy(i, b).wait()
            write_copy(i, b).start()
            b2 = (b + 2) % NBUF

            @pl.when(i >= 2)
            def _():
                write_copy(i - 2, b2).wait()

            @pl.when(i + 2 < N_CHUNKS)
            def _():
                gather_copy(i + 2, b2).start()

        return carry

    lax.fori_loop(0, N_CHUNKS // NBUF, group, 0)

    write_copy(N_CHUNKS - 2, (N_CHUNKS - 2) % NBUF).wait()
    write_copy(N_CHUNKS - 1, (N_CHUNKS - 1) % NBUF).wait()


def kernel(token_ids, weights):
    flat_ids = token_ids.reshape(B_TOTAL)
    out = _sc_gather(flat_ids, weights)
    return out.reshape(BATCH, HIST_LEN, EMBEDDING_DIM)
